# baseline (speedup 1.0000x reference)
import jax
import jax.numpy as jnp
from jax.experimental import pallas as pl
from jax.experimental.pallas import tpu as pltpu


def _standardize_block(len_ref, x_ref, o_ref):
    b = pl.program_id(0)
    L = len_ref[b]
    Lf = L.astype(jnp.float32)
    x = x_ref[0]
    t_ids = jax.lax.broadcasted_iota(jnp.int32, (x.shape[0], 1), 0)
    mask = t_ids < L
    xm = jnp.where(mask, x, 0.0)
    mean = jnp.sum(xm, axis=0, keepdims=True) / Lf
    d = jnp.where(mask, x - mean, 0.0)
    var = jnp.sum(d * d, axis=0, keepdims=True) / (Lf - 1.0)
    o_ref[0] = (x - mean) * jax.lax.rsqrt(var)


def kernel(sequence, lengths):
    B, T, D = sequence.shape
    D_BLK = 256
    grid = (B, D // D_BLK)
    return pl.pallas_call(
        _standardize_block,
        grid=grid,
        in_specs=[
            pl.BlockSpec(memory_space=pltpu.SMEM),
            pl.BlockSpec((1, T, D_BLK), lambda b, j: (b, 0, j)),
        ],
        out_specs=pl.BlockSpec((1, T, D_BLK), lambda b, j: (b, 0, j)),
        out_shape=jax.ShapeDtypeStruct((B, T, D), sequence.dtype),
        compiler_params=pltpu.CompilerParams(
            dimension_semantics=("parallel", "parallel"),
        ),
    )(lengths.astype(jnp.int32), sequence)


# speedup vs baseline: 1.1894x; 1.1894x over previous
import jax
import jax.numpy as jnp
from jax.experimental import pallas as pl
from jax.experimental.pallas import tpu as pltpu


def _standardize_block(len_ref, x_ref, o_ref):
    b = pl.program_id(0)
    L = len_ref[b]
    Lf = L.astype(jnp.float32)
    x = x_ref[0]
    t_ids = jax.lax.broadcasted_iota(jnp.int32, (x.shape[0], 1), 0)
    mask = t_ids < L
    xm = jnp.where(mask, x, 0.0)
    mean = jnp.sum(xm, axis=0, keepdims=True) / Lf
    d = jnp.where(mask, x - mean, 0.0)
    var = jnp.sum(d * d, axis=0, keepdims=True) / (Lf - 1.0)
    o_ref[0] = (x - mean) * jax.lax.rsqrt(var)


def kernel(sequence, lengths):
    B, T, D = sequence.shape
    D_BLK = 1024
    grid = (B, D // D_BLK)
    return pl.pallas_call(
        _standardize_block,
        grid=grid,
        in_specs=[
            pl.BlockSpec(memory_space=pltpu.SMEM),
            pl.BlockSpec((1, T, D_BLK), lambda b, j: (b, 0, j)),
        ],
        out_specs=pl.BlockSpec((1, T, D_BLK), lambda b, j: (b, 0, j)),
        out_shape=jax.ShapeDtypeStruct((B, T, D), sequence.dtype),
        compiler_params=pltpu.CompilerParams(
            dimension_semantics=("parallel", "parallel"),
        ),
    )(lengths.astype(jnp.int32), sequence)


# speedup vs baseline: 1.2309x; 1.0349x over previous
import jax
import jax.numpy as jnp
from jax.experimental import pallas as pl
from jax.experimental.pallas import tpu as pltpu


def _standardize_block(len_ref, x_ref, o_ref):
    b = pl.program_id(0)
    L = len_ref[b]
    Lf = L.astype(jnp.float32)
    x = x_ref[0]
    t_ids = jax.lax.broadcasted_iota(jnp.int32, (x.shape[0], 1), 0)
    xm = jnp.where(t_ids < L, x, 0.0)
    s1 = jnp.sum(xm, axis=0, keepdims=True)
    s2 = jnp.sum(xm * xm, axis=0, keepdims=True)
    mean = s1 / Lf
    var = (s2 - Lf * mean * mean) / (Lf - 1.0)
    inv = jax.lax.rsqrt(var)
    o_ref[0] = (x - mean) * inv


def kernel(sequence, lengths):
    B, T, D = sequence.shape
    D_BLK = 1024
    grid = (B, D // D_BLK)
    return pl.pallas_call(
        _standardize_block,
        grid=grid,
        in_specs=[
            pl.BlockSpec(memory_space=pltpu.SMEM),
            pl.BlockSpec((1, T, D_BLK), lambda b, j: (b, 0, j)),
        ],
        out_specs=pl.BlockSpec((1, T, D_BLK), lambda b, j: (b, 0, j)),
        out_shape=jax.ShapeDtypeStruct((B, T, D), sequence.dtype),
        compiler_params=pltpu.CompilerParams(
            dimension_semantics=("parallel", "parallel"),
        ),
    )(lengths.astype(jnp.int32), sequence)
